# Initial kernel scaffold; baseline (speedup 1.0000x reference)
import jax
import jax.numpy as jnp
from jax.experimental import pallas as pl


def kernel(use_small_u, small_scales, large_scales, start_u):
    raise NotImplementedError("write your pallas kernel here")


# speedup vs baseline: 1.0414x; 1.0414x over previous
import functools

import jax
import jax.numpy as jnp
from jax import lax
from jax.experimental import pallas as pl
from jax.experimental.pallas import tpu as pltpu
from jax.experimental.pallas import tpu_sc as plsc

_SEQ = 4096
_BATCH = 16
_NB = 4
_HALF = _SEQ // 2
_LANES = 16


def _sc_body(use_hbm, small_hbm, large_hbm, start_hbm,
             tmask_out, cmask_out, pos_out,
             use_v, small_v, large_v, start_v, tbuf, cbuf, pbuf):
    c = lax.axis_index("c")
    s = lax.axis_index("s")
    row = s
    col0 = c * _HALF

    pltpu.sync_copy(use_hbm, use_v)
    pltpu.sync_copy(small_hbm, small_v)
    pltpu.sync_copy(large_hbm, large_v)
    pltpu.sync_copy(start_hbm, start_v)

    lanes = lax.iota(jnp.int32, _LANES)
    gidx = row * _NB + (lanes & (_NB - 1))
    u = plsc.load_gather(use_v, [gidx])
    sml = plsc.load_gather(small_v, [gidx])
    lrg = plsc.load_gather(large_v, [gidx])
    su = plsc.load_gather(start_v, [gidx])
    sc = jnp.where(u < jnp.float32(0.5), sml, lrg)
    ln = jnp.maximum((sc * jnp.float32(_SEQ)).astype(jnp.int32), 1)
    mx = jnp.maximum(_SEQ - ln, 0)
    st = (su * (mx.astype(jnp.float32) + jnp.float32(1.0))).astype(jnp.int32)
    en = jnp.minimum(st + ln, _SEQ)

    key = jnp.where(lanes < _NB, st, jnp.int32(2 ** 31 - 1))
    ks, vs = plsc.sort_key_val(key, en)
    ss = [ks[i] for i in range(_NB)]
    ee = [vs[i] for i in range(_NB)]

    a0, b0 = ss[0], ee[0]
    a1 = jnp.maximum(ss[1], b0); b1 = jnp.maximum(ee[1], b0)
    a2 = jnp.maximum(ss[2], b1); b2 = jnp.maximum(ee[2], b1)
    a3 = jnp.maximum(ss[3], b2); b3 = jnp.maximum(ee[3], b2)
    c1 = b0 - a0
    c2 = c1 + (b1 - a1)
    c3 = c2 + (b2 - a2)
    c4 = c3 + (b3 - a3)
    g1 = a1 - b0
    g2 = a2 - b1
    g3 = a3 - b2
    g4 = _SEQ - b3

    def chunk(q, carry):
        j = col0 + q * _LANES + lanes
        v = j + a0
        v = v + jnp.where(j >= c1, g1, 0)
        v = v + jnp.where(j >= c2, g2, 0)
        v = v + jnp.where(j >= c3, g3, 0)
        v = v + jnp.where(j >= c4, g4, 0)
        v = jnp.minimum(v, _SEQ)
        t = ((j >= a0) & (j < b0)) | ((j >= a1) & (j < b1)) \
            | ((j >= a2) & (j < b2)) | ((j >= a3) & (j < b3))
        ti = jnp.where(t, 1, 0)
        off = q * _LANES
        pbuf[pl.ds(off, _LANES)] = v
        tbuf[pl.ds(off, _LANES)] = ti
        cbuf[pl.ds(off, _LANES)] = 1 - ti
        return carry

    lax.fori_loop(0, _HALF // _LANES, chunk, 0)

    pltpu.sync_copy(pbuf, pos_out.at[row, pl.ds(col0, _HALF)])
    pltpu.sync_copy(tbuf, tmask_out.at[row, pl.ds(col0, _HALF)])
    pltpu.sync_copy(cbuf, cmask_out.at[row, pl.ds(col0, _HALF)])


_mesh = plsc.VectorSubcoreMesh(core_axis_name="c", subcore_axis_name="s")

_span_paint = pl.kernel(
    _sc_body,
    out_type=[
        jax.ShapeDtypeStruct((_BATCH, _SEQ), jnp.int32),
        jax.ShapeDtypeStruct((_BATCH, _SEQ), jnp.int32),
        jax.ShapeDtypeStruct((_BATCH, _SEQ), jnp.int32),
    ],
    mesh=_mesh,
    scratch_types=[
        pltpu.VMEM((_BATCH * _NB,), jnp.float32),
        pltpu.VMEM((_BATCH * _NB,), jnp.float32),
        pltpu.VMEM((_BATCH * _NB,), jnp.float32),
        pltpu.VMEM((_BATCH * _NB,), jnp.float32),
        pltpu.VMEM((_HALF,), jnp.int32),
        pltpu.VMEM((_HALF,), jnp.int32),
        pltpu.VMEM((_HALF,), jnp.int32),
    ],
    compiler_params=pltpu.CompilerParams(needs_layout_passes=False),
    name="span_mask_sc",
)


def kernel(use_small_u, small_scales, large_scales, start_u):
    tmask_i32, cmask_i32, positions = _span_paint(
        use_small_u, small_scales, large_scales, start_u)
    return (cmask_i32.astype(jnp.bool_),
            tmask_i32.astype(jnp.bool_),
            positions)


# speedup vs baseline: 1.1758x; 1.1290x over previous
import functools

import jax
import jax.numpy as jnp
from jax import lax
from jax.experimental import pallas as pl
from jax.experimental.pallas import tpu as pltpu
from jax.experimental.pallas import tpu_sc as plsc

_SEQ = 4096
_BATCH = 16
_NB = 4
_HALF = _SEQ // 2
_LANES = 16


def _span_ends(u, sml, lrg, su):
    sc = jnp.where(u < jnp.float32(0.5), sml, lrg)
    ln = jnp.maximum((sc * jnp.float32(_SEQ)).astype(jnp.int32), 1)
    mx = jnp.maximum(_SEQ - ln, 0)
    st = (su * (mx.astype(jnp.float32) + jnp.float32(1.0))).astype(jnp.int32)
    en = jnp.minimum(st + ln, _SEQ)
    return st, en


def _sc_body(use_hbm, small_hbm, large_hbm, start_hbm, pos_out,
             use_v, small_v, large_v, start_v, pbuf, sem):
    c = lax.axis_index("c")
    s = lax.axis_index("s")
    row = s
    col0 = c * _HALF

    copies = [pltpu.async_copy(src, dst, sem) for src, dst in (
        (use_hbm, use_v), (small_hbm, small_v),
        (large_hbm, large_v), (start_hbm, start_v))]
    for cp in copies:
        cp.wait()

    lanes = lax.iota(jnp.int32, _LANES)
    gidx = row * _NB + (lanes & (_NB - 1))
    u = plsc.load_gather(use_v, [gidx])
    sml = plsc.load_gather(small_v, [gidx])
    lrg = plsc.load_gather(large_v, [gidx])
    su = plsc.load_gather(start_v, [gidx])
    st, en = _span_ends(u, sml, lrg, su)

    key = jnp.where(lanes < _NB, st, jnp.int32(2 ** 31 - 1))
    ks, vs = plsc.sort_key_val(key, en)
    ss = [ks[i] for i in range(_NB)]
    ee = [vs[i] for i in range(_NB)]

    a0, b0 = ss[0], ee[0]
    a1 = jnp.maximum(ss[1], b0); b1 = jnp.maximum(ee[1], b0)
    a2 = jnp.maximum(ss[2], b1); b2 = jnp.maximum(ee[2], b1)
    a3 = jnp.maximum(ss[3], b2); b3 = jnp.maximum(ee[3], b2)
    c1 = b0 - a0
    c2 = c1 + (b1 - a1)
    c3 = c2 + (b2 - a2)
    c4 = c3 + (b3 - a3)
    g1 = a1 - b0
    g2 = a2 - b1
    g3 = a3 - b2
    g4 = _SEQ - b3

    _UNROLL = 4

    def chunk(q, carry):
        base = col0 + q * (_LANES * _UNROLL)
        for r in range(_UNROLL):
            j = base + r * _LANES + lanes
            v = j + a0
            v = v + jnp.where(j >= c1, g1, 0)
            v = v + jnp.where(j >= c2, g2, 0)
            v = v + jnp.where(j >= c3, g3, 0)
            v = v + jnp.where(j >= c4, g4, 0)
            v = jnp.minimum(v, _SEQ)
            pbuf[pl.ds(q * (_LANES * _UNROLL) + r * _LANES, _LANES)] = v
        return carry

    lax.fori_loop(0, _HALF // (_LANES * _UNROLL), chunk, 0)

    pltpu.sync_copy(pbuf, pos_out.at[row, pl.ds(col0, _HALF)])


_sc_positions = pl.kernel(
    _sc_body,
    out_type=jax.ShapeDtypeStruct((_BATCH, _SEQ), jnp.int32),
    mesh=plsc.VectorSubcoreMesh(core_axis_name="c", subcore_axis_name="s"),
    scratch_types=[
        pltpu.VMEM((_BATCH * _NB,), jnp.float32),
        pltpu.VMEM((_BATCH * _NB,), jnp.float32),
        pltpu.VMEM((_BATCH * _NB,), jnp.float32),
        pltpu.VMEM((_BATCH * _NB,), jnp.float32),
        pltpu.VMEM((_HALF,), jnp.int32),
        pltpu.SemaphoreType.DMA,
    ],
    compiler_params=pltpu.CompilerParams(needs_layout_passes=False),
    name="span_positions_sc",
)


def _tc_body(use_ref, small_ref, large_ref, start_ref, cmask_ref, tmask_ref):
    st, en = _span_ends(use_ref[...], small_ref[...], large_ref[...],
                        start_ref[...])
    pos = lax.broadcasted_iota(jnp.int32, (_BATCH, _SEQ), 1)
    m = None
    for k in range(_NB):
        term = (pos >= st[:, k:k + 1]) & (pos < en[:, k:k + 1])
        m = term if m is None else m | term
    tmask_ref[...] = m
    cmask_ref[...] = jnp.logical_not(m)


_tc_masks = pl.pallas_call(
    _tc_body,
    out_shape=[
        jax.ShapeDtypeStruct((_BATCH, _SEQ), jnp.bool_),
        jax.ShapeDtypeStruct((_BATCH, _SEQ), jnp.bool_),
    ],
    name="span_masks_tc",
)


def kernel(use_small_u, small_scales, large_scales, start_u):
    rs = (_BATCH, _NB)
    positions = _sc_positions(use_small_u, small_scales,
                              large_scales, start_u)
    cmask, tmask = _tc_masks(use_small_u.reshape(rs),
                             small_scales.reshape(rs),
                             large_scales.reshape(rs),
                             start_u.reshape(rs))
    return (cmask, tmask, positions)


# speedup vs baseline: 1.2248x; 1.0417x over previous
import functools

import jax
import jax.numpy as jnp
from jax import lax
from jax.experimental import pallas as pl
from jax.experimental.pallas import tpu as pltpu
from jax.experimental.pallas import tpu_sc as plsc

_SEQ = 4096
_BATCH = 16
_NB = 4
_HALF = _SEQ // 2
_LANES = 16


def _span_ends(u, sml, lrg, su):
    sc = jnp.where(u < jnp.float32(0.5), sml, lrg)
    ln = jnp.maximum((sc * jnp.float32(_SEQ)).astype(jnp.int32), 1)
    mx = jnp.maximum(_SEQ - ln, 0)
    st = (su * (mx.astype(jnp.float32) + jnp.float32(1.0))).astype(jnp.int32)
    en = jnp.minimum(st + ln, _SEQ)
    return st, en


def _sc_body(params_hbm, pos_out, params_v, pbuf, sem):
    c = lax.axis_index("c")
    s = lax.axis_index("s")
    row = s
    col0 = c * _HALF

    pltpu.async_copy(params_hbm, params_v, sem).wait()

    lanes = lax.iota(jnp.int32, _LANES)
    gidx = row * _NB + (lanes & (_NB - 1))

    def grab(q):
        return plsc.load_gather(params_v, [jnp.broadcast_to(q, (16,)), gidx])

    st, en = _span_ends(grab(0), grab(1), grab(2), grab(3))

    key = jnp.where(lanes < _NB, st, jnp.int32(2 ** 31 - 1))
    ks, vs = plsc.sort_key_val(key, en)
    ss = [ks[i] for i in range(_NB)]
    ee = [vs[i] for i in range(_NB)]

    a0, b0 = ss[0], ee[0]
    a1 = jnp.maximum(ss[1], b0); b1 = jnp.maximum(ee[1], b0)
    a2 = jnp.maximum(ss[2], b1); b2 = jnp.maximum(ee[2], b1)
    a3 = jnp.maximum(ss[3], b2); b3 = jnp.maximum(ee[3], b2)
    c1 = b0 - a0
    c2 = c1 + (b1 - a1)
    c3 = c2 + (b2 - a2)
    c4 = c3 + (b3 - a3)
    g1 = a1 - b0
    g2 = a2 - b1
    g3 = a3 - b2
    g4 = _SEQ - b3

    _UNROLL = 2

    def chunk(q, carry):
        base = q * (_LANES * _UNROLL)
        for r in range(_UNROLL):
            j = col0 + base + r * _LANES + lanes
            v = j + a0
            v = v + jnp.where(j >= c1, g1, 0)
            v = v + jnp.where(j >= c2, g2, 0)
            v = v + jnp.where(j >= c3, g3, 0)
            v = v + jnp.where(j >= c4, g4, 0)
            v = jnp.minimum(v, _SEQ)
            pbuf[pl.ds(base + r * _LANES, _LANES)] = v
        return carry

    lax.fori_loop(0, _HALF // (_LANES * _UNROLL), chunk, 0)

    pltpu.sync_copy(pbuf, pos_out.at[row, pl.ds(col0, _HALF)])


_sc_positions = pl.kernel(
    _sc_body,
    out_type=jax.ShapeDtypeStruct((_BATCH, _SEQ), jnp.int32),
    mesh=plsc.VectorSubcoreMesh(core_axis_name="c", subcore_axis_name="s"),
    scratch_types=[
        pltpu.VMEM((_NB, _BATCH * _NB), jnp.float32),
        pltpu.VMEM((_HALF,), jnp.int32),
        pltpu.SemaphoreType.DMA,
    ],
    compiler_params=pltpu.CompilerParams(needs_layout_passes=False),
    name="span_positions_sc",
)


def _tc_body(params_ref, cmask_ref, tmask_ref):
    st, en = _span_ends(params_ref[0], params_ref[1], params_ref[2],
                        params_ref[3])
    pos = lax.broadcasted_iota(jnp.int32, (_BATCH, _SEQ), 1)
    m = None
    for k in range(_NB):
        term = (pos >= st[:, k:k + 1]) & (pos < en[:, k:k + 1])
        m = term if m is None else m | term
    tmask_ref[...] = m
    cmask_ref[...] = jnp.logical_not(m)


_tc_masks = pl.pallas_call(
    _tc_body,
    out_shape=[
        jax.ShapeDtypeStruct((_BATCH, _SEQ), jnp.bool_),
        jax.ShapeDtypeStruct((_BATCH, _SEQ), jnp.bool_),
    ],
    name="span_masks_tc",
)


def kernel(use_small_u, small_scales, large_scales, start_u):
    params = jnp.stack([use_small_u, small_scales, large_scales, start_u])
    positions = _sc_positions(params)
    cmask, tmask = _tc_masks(params.reshape(_NB, _BATCH, _NB))
    return (cmask, tmask, positions)
